# Initial kernel scaffold; baseline (speedup 1.0000x reference)
import jax
import jax.numpy as jnp
from jax.experimental import pallas as pl


def kernel(x, edge_index, W1l, b1l, W1r, W2l, b2l, W2r):
    raise NotImplementedError("write your pallas kernel here")


# speedup vs baseline: 11.3647x; 11.3647x over previous
import functools

import jax
import jax.numpy as jnp
from jax import lax
from jax.experimental import pallas as pl
from jax.experimental.pallas import tpu as pltpu
from jax.experimental.pallas import tpu_sc as plsc

N_NODES = 10000
N_PAD = 10240
D_IN = 128
D_HID = 16
D_OUT = 40
ROWS_PER_TILE = N_PAD // 16
CHUNK = 128
ROW_BLK = 512
N_TC_BLKS = N_PAD // ROW_BLK


def _sc_segsum_body(with_deg, feat, src2d, dst2d, *rest):
    if with_deg:
        (agg_out, deg_out, src_v, dst_v, msgs, ones_v, zbuf, dzbuf,
         agg_sh, deg_sh, sem) = rest
    else:
        agg_out, src_v, dst_v, msgs, zbuf, agg_sh, sem = rest
    cid = lax.axis_index("c")
    sid = lax.axis_index("s")
    wid = cid * 16 + sid
    n_chunks = src2d.shape[0] // 32

    def _zrow(i, _):
        zbuf[i, :] = jnp.zeros((16,), jnp.float32)
        return 0
    lax.fori_loop(0, ROWS_PER_TILE, _zrow, 0)
    pltpu.sync_copy(zbuf, agg_sh.at[pl.ds(sid * ROWS_PER_TILE, ROWS_PER_TILE)])
    if with_deg:
        def _zrow1(i, _):
            dzbuf[pl.ds(i * 16, 16)] = jnp.zeros((16,), jnp.float32)
            return 0
        lax.fori_loop(0, ROWS_PER_TILE // 16, _zrow1, 0)
        pltpu.sync_copy(dzbuf,
                        deg_sh.at[pl.ds(sid * ROWS_PER_TILE, ROWS_PER_TILE)])
        for j in range(CHUNK // 16):
            ones_v[pl.ds(j * 16, 16)] = jnp.ones((16,), jnp.float32)
    plsc.subcore_barrier()

    pltpu.sync_copy(src2d.at[pl.ds(wid * n_chunks, n_chunks)], src_v)
    pltpu.sync_copy(dst2d.at[pl.ds(wid * n_chunks, n_chunks)], dst_v)

    def _chunk(j, _):
        pltpu.async_copy(feat.at[src_v.at[j]], msgs, sem).wait()
        pltpu.sync_copy(msgs, agg_sh.at[dst_v.at[j]], add=True)
        if with_deg:
            pltpu.sync_copy(ones_v, deg_sh.at[dst_v.at[j]], add=True)
        return 0
    lax.fori_loop(0, n_chunks, _chunk, 0)
    plsc.subcore_barrier()

    sl = pl.ds(sid * ROWS_PER_TILE, ROWS_PER_TILE)
    pltpu.sync_copy(agg_sh.at[sl], agg_out.at[cid, sl])
    if with_deg:
        pltpu.sync_copy(deg_sh.at[sl], deg_out.at[cid, sl])


def _mm_body(x_ref, wl_ref, wr_ref, xl_ref, xr_ref):
    xb = x_ref[...]
    xl_ref[...] = jnp.dot(xb, wl_ref[...], preferred_element_type=jnp.float32)
    xr_ref[...] = jnp.dot(xb, wr_ref[...], preferred_element_type=jnp.float32)


def _mid_body(aggp_ref, degp_ref, xr_ref, b1l_ref, h_ref, dinv_ref):
    deg = jnp.maximum(degp_ref[0] + degp_ref[1], 1.0)
    dinv = 1.0 / deg
    t = (aggp_ref[0] + aggp_ref[1]) * dinv + b1l_ref[...] + xr_ref[...]
    nrm = jnp.sqrt(jnp.sum(t * t, axis=1, keepdims=True))
    t = t / jnp.maximum(nrm, 1e-12)
    h_ref[...] = jnp.maximum(t, 0.0)
    dinv_ref[...] = dinv


def _out_body(aggp_ref, dinv_ref, h_ref, w2l_ref, w2r_ref, b2l_ref, o_ref):
    agg2 = (aggp_ref[0] + aggp_ref[1]) * dinv_ref[...]
    z = (jnp.dot(agg2, w2l_ref[...], preferred_element_type=jnp.float32)
         + jnp.dot(h_ref[...], w2r_ref[...], preferred_element_type=jnp.float32)
         + b2l_ref[...])
    z = z - jnp.max(z, axis=1, keepdims=True)
    e = jnp.exp(z)
    o_ref[...] = e / jnp.sum(e, axis=1, keepdims=True)


def kernel(x, edge_index, W1l, b1l, W1r, W2l, b2l, W2r):
    src = edge_index[0]
    dst = edge_index[1]
    e_total = src.shape[0]
    n_chunks_tile = -(-e_total // (32 * CHUNK * 8)) * 8
    e_pad = 32 * CHUNK * n_chunks_tile
    pad_n = e_pad - e_total
    ar = jnp.arange(pad_n, dtype=jnp.int32)
    src_p = jnp.concatenate([src, ar % N_NODES]).reshape(-1, CHUNK)
    dst_p = jnp.concatenate([dst, N_NODES + ar % (N_PAD - N_NODES)]
                            ).reshape(-1, CHUNK)

    xp = jnp.pad(x, ((0, N_PAD - x.shape[0]), (0, 0)))

    xl, xr = pl.pallas_call(
        _mm_body,
        grid=(N_TC_BLKS,),
        in_specs=[
            pl.BlockSpec((ROW_BLK, D_IN), lambda i: (i, 0)),
            pl.BlockSpec((D_IN, D_HID), lambda i: (0, 0)),
            pl.BlockSpec((D_IN, D_HID), lambda i: (0, 0)),
        ],
        out_specs=[
            pl.BlockSpec((ROW_BLK, D_HID), lambda i: (i, 0)),
            pl.BlockSpec((ROW_BLK, D_HID), lambda i: (i, 0)),
        ],
        out_shape=[
            jax.ShapeDtypeStruct((N_PAD, D_HID), jnp.float32),
            jax.ShapeDtypeStruct((N_PAD, D_HID), jnp.float32),
        ],
    )(xp, W1l.T, W1r.T)

    mesh = plsc.VectorSubcoreMesh(core_axis_name="c", subcore_axis_name="s")

    sc1 = pl.kernel(
        functools.partial(_sc_segsum_body, True),
        mesh=mesh,
        out_type=[
            jax.ShapeDtypeStruct((2, N_PAD, D_HID), jnp.float32),
            jax.ShapeDtypeStruct((2, N_PAD), jnp.float32),
        ],
        scratch_types=[
            pltpu.VMEM((n_chunks_tile, CHUNK), jnp.int32),
            pltpu.VMEM((n_chunks_tile, CHUNK), jnp.int32),
            pltpu.VMEM((CHUNK, D_HID), jnp.float32),
            pltpu.VMEM((CHUNK,), jnp.float32),
            pltpu.VMEM((ROWS_PER_TILE, D_HID), jnp.float32),
            pltpu.VMEM((ROWS_PER_TILE,), jnp.float32),
            pltpu.VMEM_SHARED((N_PAD, D_HID), jnp.float32),
            pltpu.VMEM_SHARED((N_PAD,), jnp.float32),
            pltpu.SemaphoreType.DMA,
        ],
        compiler_params=pltpu.CompilerParams(use_tc_tiling_on_sc=False),
    )
    aggp1, degp = sc1(xl, src_p, dst_p)

    h, dinv = pl.pallas_call(
        _mid_body,
        grid=(N_TC_BLKS,),
        in_specs=[
            pl.BlockSpec((2, ROW_BLK, D_HID), lambda i: (0, i, 0)),
            pl.BlockSpec((2, ROW_BLK, 1), lambda i: (0, i, 0)),
            pl.BlockSpec((ROW_BLK, D_HID), lambda i: (i, 0)),
            pl.BlockSpec((1, D_HID), lambda i: (0, 0)),
        ],
        out_specs=[
            pl.BlockSpec((ROW_BLK, D_HID), lambda i: (i, 0)),
            pl.BlockSpec((ROW_BLK, 1), lambda i: (i, 0)),
        ],
        out_shape=[
            jax.ShapeDtypeStruct((N_PAD, D_HID), jnp.float32),
            jax.ShapeDtypeStruct((N_PAD, 1), jnp.float32),
        ],
    )(aggp1, degp.reshape(2, N_PAD, 1), xr, b1l.reshape(1, D_HID))

    sc2 = pl.kernel(
        functools.partial(_sc_segsum_body, False),
        mesh=mesh,
        out_type=[jax.ShapeDtypeStruct((2, N_PAD, D_HID), jnp.float32)],
        scratch_types=[
            pltpu.VMEM((n_chunks_tile, CHUNK), jnp.int32),
            pltpu.VMEM((n_chunks_tile, CHUNK), jnp.int32),
            pltpu.VMEM((CHUNK, D_HID), jnp.float32),
            pltpu.VMEM((ROWS_PER_TILE, D_HID), jnp.float32),
            pltpu.VMEM_SHARED((N_PAD, D_HID), jnp.float32),
            pltpu.SemaphoreType.DMA,
        ],
        compiler_params=pltpu.CompilerParams(use_tc_tiling_on_sc=False),
    )
    aggp2 = sc2(h, src_p, dst_p)
    if isinstance(aggp2, (list, tuple)):
        aggp2 = aggp2[0]

    out = pl.pallas_call(
        _out_body,
        grid=(N_TC_BLKS,),
        in_specs=[
            pl.BlockSpec((2, ROW_BLK, D_HID), lambda i: (0, i, 0)),
            pl.BlockSpec((ROW_BLK, 1), lambda i: (i, 0)),
            pl.BlockSpec((ROW_BLK, D_HID), lambda i: (i, 0)),
            pl.BlockSpec((D_HID, D_OUT), lambda i: (0, 0)),
            pl.BlockSpec((D_HID, D_OUT), lambda i: (0, 0)),
            pl.BlockSpec((1, D_OUT), lambda i: (0, 0)),
        ],
        out_specs=pl.BlockSpec((ROW_BLK, D_OUT), lambda i: (i, 0)),
        out_shape=jax.ShapeDtypeStruct((N_PAD, D_OUT), jnp.float32),
    )(aggp2, dinv, h, W2l.T, W2r.T, b2l.reshape(1, D_OUT))

    return out[:N_NODES]


# speedup vs baseline: 17.5391x; 1.5433x over previous
import functools

import jax
import jax.numpy as jnp
from jax import lax
from jax.experimental import pallas as pl
from jax.experimental.pallas import tpu as pltpu
from jax.experimental.pallas import tpu_sc as plsc

N_NODES = 10000
N_PAD = 10240
D_IN = 128
D_HID = 16
D_OUT = 40
ROWS_PER_TILE = N_PAD // 16
CHUNK = 128
NBUF = 4
ROW_BLK = 512
N_TC_BLKS = N_PAD // ROW_BLK


def _sc_segsum_body(with_deg, feat, src2d, dst2d, *rest):
    if with_deg:
        (agg_out, deg_out, src_v, dst_v, msgs, ones_v, zbuf, dzbuf,
         agg_sh, deg_sh, sem) = rest
    else:
        agg_out, src_v, dst_v, msgs, zbuf, agg_sh, sem = rest
    cid = lax.axis_index("c")
    sid = lax.axis_index("s")
    wid = cid * 16 + sid
    n_chunks = src2d.shape[0] // 32

    def _zrow(i, _):
        zbuf[i, :] = jnp.zeros((16,), jnp.float32)
        return 0
    lax.fori_loop(0, ROWS_PER_TILE, _zrow, 0)
    pltpu.sync_copy(zbuf, agg_sh.at[pl.ds(sid * ROWS_PER_TILE, ROWS_PER_TILE)])
    if with_deg:
        def _zrow1(i, _):
            dzbuf[pl.ds(i * 16, 16)] = jnp.zeros((16,), jnp.float32)
            return 0
        lax.fori_loop(0, ROWS_PER_TILE // 16, _zrow1, 0)
        pltpu.sync_copy(dzbuf,
                        deg_sh.at[pl.ds(sid * ROWS_PER_TILE, ROWS_PER_TILE)])
        for j in range(CHUNK // 16):
            ones_v[pl.ds(j * 16, 16)] = jnp.ones((16,), jnp.float32)
    plsc.subcore_barrier()

    pltpu.sync_copy(src2d.at[pl.ds(wid * n_chunks, n_chunks)], src_v)
    pltpu.sync_copy(dst2d.at[pl.ds(wid * n_chunks, n_chunks)], dst_v)

    for b in range(NBUF):
        pltpu.async_copy(feat.at[src_v.at[b]], msgs.at[b], sem.at[b])

    def _group(g, _):
        j = g * NBUF
        for b in range(NBUF):
            pltpu.make_async_copy(feat.at[src_v.at[j + b]], msgs.at[b],
                                  sem.at[b]).wait()
            pltpu.sync_copy(msgs.at[b], agg_sh.at[dst_v.at[j + b]], add=True)
            if with_deg:
                pltpu.sync_copy(ones_v, deg_sh.at[dst_v.at[j + b]], add=True)

            @pl.when(j + b + NBUF < n_chunks)
            def _():
                pltpu.async_copy(feat.at[src_v.at[j + b + NBUF]], msgs.at[b],
                                 sem.at[b])
        return 0
    lax.fori_loop(0, n_chunks // NBUF, _group, 0)
    plsc.subcore_barrier()

    sl = pl.ds(sid * ROWS_PER_TILE, ROWS_PER_TILE)
    pltpu.sync_copy(agg_sh.at[sl], agg_out.at[cid, sl])
    if with_deg:
        pltpu.sync_copy(deg_sh.at[sl], deg_out.at[cid, sl])


def _mm_body(x_ref, wl_ref, wr_ref, xl_ref, xr_ref):
    xb = x_ref[...]
    xl_ref[...] = jnp.dot(xb, wl_ref[...], preferred_element_type=jnp.float32)
    xr_ref[...] = jnp.dot(xb, wr_ref[...], preferred_element_type=jnp.float32)


def _mid_body(aggp_ref, degp_ref, xr_ref, b1l_ref, h_ref, dinv_ref):
    deg = jnp.maximum(degp_ref[0] + degp_ref[1], 1.0)
    dinv = 1.0 / deg
    t = (aggp_ref[0] + aggp_ref[1]) * dinv + b1l_ref[...] + xr_ref[...]
    nrm = jnp.sqrt(jnp.sum(t * t, axis=1, keepdims=True))
    t = t / jnp.maximum(nrm, 1e-12)
    h_ref[...] = jnp.maximum(t, 0.0)
    dinv_ref[...] = dinv


def _out_body(aggp_ref, dinv_ref, h_ref, w2l_ref, w2r_ref, b2l_ref, o_ref):
    agg2 = (aggp_ref[0] + aggp_ref[1]) * dinv_ref[...]
    z = (jnp.dot(agg2, w2l_ref[...], preferred_element_type=jnp.float32)
         + jnp.dot(h_ref[...], w2r_ref[...], preferred_element_type=jnp.float32)
         + b2l_ref[...])
    z = z - jnp.max(z, axis=1, keepdims=True)
    e = jnp.exp(z)
    o_ref[...] = e / jnp.sum(e, axis=1, keepdims=True)


def kernel(x, edge_index, W1l, b1l, W1r, W2l, b2l, W2r):
    src = edge_index[0]
    dst = edge_index[1]
    e_total = src.shape[0]
    n_chunks_tile = -(-e_total // (32 * CHUNK * 8)) * 8
    e_pad = 32 * CHUNK * n_chunks_tile
    pad_n = e_pad - e_total
    ar = jnp.arange(pad_n, dtype=jnp.int32)
    src_p = jnp.concatenate([src, ar % N_NODES]).reshape(-1, CHUNK)
    dst_p = jnp.concatenate([dst, N_NODES + ar % (N_PAD - N_NODES)]
                            ).reshape(-1, CHUNK)

    xp = jnp.pad(x, ((0, N_PAD - x.shape[0]), (0, 0)))

    xl, xr = pl.pallas_call(
        _mm_body,
        grid=(N_TC_BLKS,),
        in_specs=[
            pl.BlockSpec((ROW_BLK, D_IN), lambda i: (i, 0)),
            pl.BlockSpec((D_IN, D_HID), lambda i: (0, 0)),
            pl.BlockSpec((D_IN, D_HID), lambda i: (0, 0)),
        ],
        out_specs=[
            pl.BlockSpec((ROW_BLK, D_HID), lambda i: (i, 0)),
            pl.BlockSpec((ROW_BLK, D_HID), lambda i: (i, 0)),
        ],
        out_shape=[
            jax.ShapeDtypeStruct((N_PAD, D_HID), jnp.float32),
            jax.ShapeDtypeStruct((N_PAD, D_HID), jnp.float32),
        ],
    )(xp, W1l.T, W1r.T)

    mesh = plsc.VectorSubcoreMesh(core_axis_name="c", subcore_axis_name="s")

    sc1 = pl.kernel(
        functools.partial(_sc_segsum_body, True),
        mesh=mesh,
        out_type=[
            jax.ShapeDtypeStruct((2, N_PAD, D_HID), jnp.float32),
            jax.ShapeDtypeStruct((2, N_PAD), jnp.float32),
        ],
        scratch_types=[
            pltpu.VMEM((n_chunks_tile, CHUNK), jnp.int32),
            pltpu.VMEM((n_chunks_tile, CHUNK), jnp.int32),
            pltpu.VMEM((NBUF, CHUNK, D_HID), jnp.float32),
            pltpu.VMEM((CHUNK,), jnp.float32),
            pltpu.VMEM((ROWS_PER_TILE, D_HID), jnp.float32),
            pltpu.VMEM((ROWS_PER_TILE,), jnp.float32),
            pltpu.VMEM_SHARED((N_PAD, D_HID), jnp.float32),
            pltpu.VMEM_SHARED((N_PAD,), jnp.float32),
            pltpu.SemaphoreType.DMA((NBUF,)),
        ],
        compiler_params=pltpu.CompilerParams(use_tc_tiling_on_sc=False),
    )
    aggp1, degp = sc1(xl, src_p, dst_p)

    h, dinv = pl.pallas_call(
        _mid_body,
        grid=(N_TC_BLKS,),
        in_specs=[
            pl.BlockSpec((2, ROW_BLK, D_HID), lambda i: (0, i, 0)),
            pl.BlockSpec((2, ROW_BLK, 1), lambda i: (0, i, 0)),
            pl.BlockSpec((ROW_BLK, D_HID), lambda i: (i, 0)),
            pl.BlockSpec((1, D_HID), lambda i: (0, 0)),
        ],
        out_specs=[
            pl.BlockSpec((ROW_BLK, D_HID), lambda i: (i, 0)),
            pl.BlockSpec((ROW_BLK, 1), lambda i: (i, 0)),
        ],
        out_shape=[
            jax.ShapeDtypeStruct((N_PAD, D_HID), jnp.float32),
            jax.ShapeDtypeStruct((N_PAD, 1), jnp.float32),
        ],
    )(aggp1, degp.reshape(2, N_PAD, 1), xr, b1l.reshape(1, D_HID))

    sc2 = pl.kernel(
        functools.partial(_sc_segsum_body, False),
        mesh=mesh,
        out_type=[jax.ShapeDtypeStruct((2, N_PAD, D_HID), jnp.float32)],
        scratch_types=[
            pltpu.VMEM((n_chunks_tile, CHUNK), jnp.int32),
            pltpu.VMEM((n_chunks_tile, CHUNK), jnp.int32),
            pltpu.VMEM((NBUF, CHUNK, D_HID), jnp.float32),
            pltpu.VMEM((ROWS_PER_TILE, D_HID), jnp.float32),
            pltpu.VMEM_SHARED((N_PAD, D_HID), jnp.float32),
            pltpu.SemaphoreType.DMA((NBUF,)),
        ],
        compiler_params=pltpu.CompilerParams(use_tc_tiling_on_sc=False),
    )
    aggp2 = sc2(h, src_p, dst_p)
    if isinstance(aggp2, (list, tuple)):
        aggp2 = aggp2[0]

    out = pl.pallas_call(
        _out_body,
        grid=(N_TC_BLKS,),
        in_specs=[
            pl.BlockSpec((2, ROW_BLK, D_HID), lambda i: (0, i, 0)),
            pl.BlockSpec((ROW_BLK, 1), lambda i: (i, 0)),
            pl.BlockSpec((ROW_BLK, D_HID), lambda i: (i, 0)),
            pl.BlockSpec((D_HID, D_OUT), lambda i: (0, 0)),
            pl.BlockSpec((D_HID, D_OUT), lambda i: (0, 0)),
            pl.BlockSpec((1, D_OUT), lambda i: (0, 0)),
        ],
        out_specs=pl.BlockSpec((ROW_BLK, D_OUT), lambda i: (i, 0)),
        out_shape=jax.ShapeDtypeStruct((N_PAD, D_OUT), jnp.float32),
    )(aggp2, dinv, h, W2l.T, W2r.T, b2l.reshape(1, D_OUT))

    return out[:N_NODES]


# speedup vs baseline: 19.4333x; 1.1080x over previous
import functools

import jax
import jax.numpy as jnp
from jax import lax
from jax.experimental import pallas as pl
from jax.experimental.pallas import tpu as pltpu
from jax.experimental.pallas import tpu_sc as plsc

N_NODES = 10000
N_PAD = 10240
D_IN = 128
D_HID = 16
D_OUT = 40
ROWS_PER_TILE = N_PAD // 16
CHUNK = 128
NBUF = 4


def _sc_segsum_body(with_deg, n_rows, feat, src2d, dst2d, *rest):
    if with_deg:
        (agg_out, deg_out, src_v, dst_v, msgs, ones_v, zbuf, dzbuf,
         agg_sh, deg_sh, sem) = rest
    else:
        agg_out, src_v, dst_v, msgs, zbuf, agg_sh, sem = rest
    cid = lax.axis_index("c")
    sid = lax.axis_index("s")
    wid = cid * 16 + sid
    q, r = divmod(n_rows, 32)
    q_main = (q // NBUF) * NBUF
    base = wid * q

    def _zrow(i, _):
        zbuf[i, :] = jnp.zeros((16,), jnp.float32)
        return 0
    lax.fori_loop(0, ROWS_PER_TILE, _zrow, 0)
    pltpu.sync_copy(zbuf, agg_sh.at[pl.ds(sid * ROWS_PER_TILE, ROWS_PER_TILE)])
    if with_deg:
        def _zrow1(i, _):
            dzbuf[pl.ds(i * 16, 16)] = jnp.zeros((16,), jnp.float32)
            return 0
        lax.fori_loop(0, ROWS_PER_TILE // 16, _zrow1, 0)
        pltpu.sync_copy(dzbuf,
                        deg_sh.at[pl.ds(sid * ROWS_PER_TILE, ROWS_PER_TILE)])
        for j in range(CHUNK // 16):
            ones_v[pl.ds(j * 16, 16)] = jnp.ones((16,), jnp.float32)
    plsc.subcore_barrier()

    pltpu.sync_copy(src2d.at[pl.ds(base, q)], src_v.at[pl.ds(0, q)])
    pltpu.sync_copy(dst2d.at[pl.ds(base, q)], dst_v.at[pl.ds(0, q)])
    if r:
        @pl.when(wid < r)
        def _():
            pltpu.sync_copy(src2d.at[pl.ds(32 * q + wid, 1)],
                            src_v.at[pl.ds(q, 1)])
            pltpu.sync_copy(dst2d.at[pl.ds(32 * q + wid, 1)],
                            dst_v.at[pl.ds(q, 1)])

    def _do_chunk(j, b):
        pltpu.make_async_copy(feat.at[src_v.at[j]], msgs.at[b],
                              sem.at[b]).wait()
        pltpu.sync_copy(msgs.at[b], agg_sh.at[dst_v.at[j]], add=True)
        if with_deg:
            pltpu.sync_copy(ones_v, deg_sh.at[dst_v.at[j]], add=True)

    for b in range(NBUF):
        pltpu.async_copy(feat.at[src_v.at[b]], msgs.at[b], sem.at[b])

    def _group(g, _):
        j = g * NBUF
        for b in range(NBUF):
            _do_chunk(j + b, b)

            @pl.when(j + b + NBUF < q_main)
            def _():
                pltpu.async_copy(feat.at[src_v.at[j + b + NBUF]], msgs.at[b],
                                 sem.at[b])
        return 0
    lax.fori_loop(0, q_main // NBUF, _group, 0)

    for t in range(q_main, q):
        pltpu.async_copy(feat.at[src_v.at[t]], msgs.at[0], sem.at[0])
        _do_chunk(t, 0)
    if r:
        @pl.when(wid < r)
        def _():
            pltpu.async_copy(feat.at[src_v.at[q]], msgs.at[0], sem.at[0])
            _do_chunk(q, 0)
    plsc.subcore_barrier()

    sl = pl.ds(sid * ROWS_PER_TILE, ROWS_PER_TILE)
    pltpu.sync_copy(agg_sh.at[sl], agg_out.at[cid, sl])
    if with_deg:
        pltpu.sync_copy(deg_sh.at[sl], deg_out.at[cid, sl])


def _make_sc_call(with_deg, n_rows):
    mesh = plsc.VectorSubcoreMesh(core_axis_name="c", subcore_axis_name="s")
    q = n_rows // 32
    if with_deg:
        out_type = [
            jax.ShapeDtypeStruct((2, N_PAD, D_HID), jnp.float32),
            jax.ShapeDtypeStruct((2, N_PAD), jnp.float32),
        ]
        scratch = [
            pltpu.VMEM((q + 1, CHUNK), jnp.int32),
            pltpu.VMEM((q + 1, CHUNK), jnp.int32),
            pltpu.VMEM((NBUF, CHUNK, D_HID), jnp.float32),
            pltpu.VMEM((CHUNK,), jnp.float32),
            pltpu.VMEM((ROWS_PER_TILE, D_HID), jnp.float32),
            pltpu.VMEM((ROWS_PER_TILE,), jnp.float32),
            pltpu.VMEM_SHARED((N_PAD, D_HID), jnp.float32),
            pltpu.VMEM_SHARED((N_PAD,), jnp.float32),
            pltpu.SemaphoreType.DMA((NBUF,)),
        ]
    else:
        out_type = [jax.ShapeDtypeStruct((2, N_PAD, D_HID), jnp.float32)]
        scratch = [
            pltpu.VMEM((q + 1, CHUNK), jnp.int32),
            pltpu.VMEM((q + 1, CHUNK), jnp.int32),
            pltpu.VMEM((NBUF, CHUNK, D_HID), jnp.float32),
            pltpu.VMEM((ROWS_PER_TILE, D_HID), jnp.float32),
            pltpu.VMEM_SHARED((N_PAD, D_HID), jnp.float32),
            pltpu.SemaphoreType.DMA((NBUF,)),
        ]
    return pl.kernel(
        functools.partial(_sc_segsum_body, with_deg, n_rows),
        mesh=mesh,
        out_type=out_type,
        scratch_types=scratch,
        compiler_params=pltpu.CompilerParams(use_tc_tiling_on_sc=False),
    )


def _mm_body(x_ref, wl_ref, wr_ref, xl_ref, xr_ref):
    xb = x_ref[...]
    xl_ref[...] = jnp.dot(xb, wl_ref[...], preferred_element_type=jnp.float32)
    xr_ref[...] = jnp.dot(xb, wr_ref[...], preferred_element_type=jnp.float32)


def _mid_body(aggp_ref, degp_ref, xr_ref, b1l_ref, h_ref, dinv_ref):
    n = xr_ref.shape[0]
    deg = jnp.maximum(degp_ref[0, :n, :] + degp_ref[1, :n, :], 1.0)
    dinv = 1.0 / deg
    t = ((aggp_ref[0, :n, :] + aggp_ref[1, :n, :]) * dinv
         + b1l_ref[...] + xr_ref[...])
    nrm = jnp.sqrt(jnp.sum(t * t, axis=1, keepdims=True))
    t = t / jnp.maximum(nrm, 1e-12)
    h_ref[...] = jnp.maximum(t, 0.0)
    dinv_ref[...] = dinv


def _out_body(aggp_ref, dinv_ref, h_ref, w2l_ref, w2r_ref, b2l_ref, o_ref):
    n = h_ref.shape[0]
    agg2 = (aggp_ref[0, :n, :] + aggp_ref[1, :n, :]) * dinv_ref[...]
    z = (jnp.dot(agg2, w2l_ref[...], preferred_element_type=jnp.float32)
         + jnp.dot(h_ref[...], w2r_ref[...], preferred_element_type=jnp.float32)
         + b2l_ref[...])
    z = z - jnp.max(z, axis=1, keepdims=True)
    e = jnp.exp(z)
    o_ref[...] = e / jnp.sum(e, axis=1, keepdims=True)


def kernel(x, edge_index, W1l, b1l, W1r, W2l, b2l, W2r):
    n = x.shape[0]
    e_total = edge_index.shape[1]
    if e_total % CHUNK:
        pad_n = CHUNK - e_total % CHUNK
        ar = jnp.arange(pad_n, dtype=jnp.int32)
        src2d = jnp.concatenate([edge_index[0], ar % n]).reshape(-1, CHUNK)
        dst2d = jnp.concatenate(
            [edge_index[1], N_NODES + ar % (N_PAD - N_NODES)]
        ).reshape(-1, CHUNK)
    else:
        src2d = edge_index[0].reshape(-1, CHUNK)
        dst2d = edge_index[1].reshape(-1, CHUNK)
    n_rows = src2d.shape[0]

    xl, xr = pl.pallas_call(
        _mm_body,
        out_shape=[
            jax.ShapeDtypeStruct((n, D_HID), jnp.float32),
            jax.ShapeDtypeStruct((n, D_HID), jnp.float32),
        ],
    )(x, W1l.T, W1r.T)

    aggp1, degp = _make_sc_call(True, n_rows)(xl, src2d, dst2d)

    h, dinv = pl.pallas_call(
        _mid_body,
        out_shape=[
            jax.ShapeDtypeStruct((n, D_HID), jnp.float32),
            jax.ShapeDtypeStruct((n, 1), jnp.float32),
        ],
    )(aggp1, degp.reshape(2, N_PAD, 1), xr, b1l.reshape(1, D_HID))

    aggp2 = _make_sc_call(False, n_rows)(h, src2d, dst2d)
    if isinstance(aggp2, (list, tuple)):
        aggp2 = aggp2[0]

    out = pl.pallas_call(
        _out_body,
        out_shape=jax.ShapeDtypeStruct((n, D_OUT), jnp.float32),
    )(aggp2, dinv, h, W2l.T, W2r.T, b2l.reshape(1, D_OUT))
    return out


# speedup vs baseline: 19.5192x; 1.0044x over previous
import functools

import jax
import jax.numpy as jnp
from jax import lax
from jax.experimental import pallas as pl
from jax.experimental.pallas import tpu as pltpu
from jax.experimental.pallas import tpu_sc as plsc

N_NODES = 10000
N_PAD = 10240
D_IN = 128
D_HID = 16
D_OUT = 40
ROWS_PER_TILE = N_PAD // 16
CHUNK = 128
NBUF = 4
NSLOT = 8


def _sc_segsum_body(with_deg, n_rows, feat, src1d, dst1d, *rest):
    if with_deg:
        (agg_out, deg_out, src_v, dst_v, msgs, ones_v, zbuf, dzbuf,
         agg_sh, deg_sh, gsem, ssem, dsem) = rest
    else:
        agg_out, src_v, dst_v, msgs, zbuf, agg_sh, gsem, ssem = rest
    cid = lax.axis_index("c")
    sid = lax.axis_index("s")
    wid = cid * 16 + sid
    q, r = divmod(n_rows, 32)
    q_main = (q // NSLOT) * NSLOT
    base = wid * q

    def _zrow(i, _):
        zbuf[i, :] = jnp.zeros((16,), jnp.float32)
        return 0
    lax.fori_loop(0, ROWS_PER_TILE, _zrow, 0)
    pltpu.sync_copy(zbuf, agg_sh.at[pl.ds(sid * ROWS_PER_TILE, ROWS_PER_TILE)])
    if with_deg:
        def _zrow1(i, _):
            dzbuf[pl.ds(i * 16, 16)] = jnp.zeros((16,), jnp.float32)
            return 0
        lax.fori_loop(0, ROWS_PER_TILE // 16, _zrow1, 0)
        pltpu.sync_copy(dzbuf,
                        deg_sh.at[pl.ds(sid * ROWS_PER_TILE, ROWS_PER_TILE)])
        for j in range(CHUNK // 16):
            ones_v[pl.ds(j * 16, 16)] = jnp.ones((16,), jnp.float32)
    plsc.subcore_barrier()

    pltpu.sync_copy(src1d.at[pl.ds(base * CHUNK, q * CHUNK)],
                    src_v.at[pl.ds(0, q * CHUNK)])
    pltpu.sync_copy(dst1d.at[pl.ds(base * CHUNK, q * CHUNK)],
                    dst_v.at[pl.ds(0, q * CHUNK)])
    if r:
        @pl.when(wid < r)
        def _():
            pltpu.sync_copy(src1d.at[pl.ds((32 * q + wid) * CHUNK, CHUNK)],
                            src_v.at[pl.ds(q * CHUNK, CHUNK)])
            pltpu.sync_copy(dst1d.at[pl.ds((32 * q + wid) * CHUNK, CHUNK)],
                            dst_v.at[pl.ds(q * CHUNK, CHUNK)])

    def _sidx(j):
        return src_v.at[pl.ds(j * CHUNK, CHUNK)]

    def _didx(j):
        return dst_v.at[pl.ds(j * CHUNK, CHUNK)]

    for b in range(NBUF):
        pltpu.async_copy(feat.at[_sidx(b)], msgs.at[b], gsem.at[b])

    def _group(g, _):
        j0 = g * NSLOT
        for b in range(NSLOT):
            j = j0 + b
            pltpu.make_async_copy(feat.at[_sidx(j)], msgs.at[b],
                                  gsem.at[b]).wait()
            pltpu.async_copy(msgs.at[b], agg_sh.at[_didx(j)], ssem.at[b],
                             add=True)
            if with_deg:
                pltpu.async_copy(ones_v, deg_sh.at[_didx(j)], dsem.at[b],
                                 add=True)
            nb = (b + NBUF) % NSLOT

            @pl.when(j + NBUF < q_main)
            def _():
                @pl.when(j + NBUF >= NSLOT)
                def _():
                    pltpu.make_async_copy(msgs.at[nb], agg_sh.at[_didx(j)],
                                          ssem.at[nb]).wait()
                    if with_deg:
                        pltpu.make_async_copy(ones_v, deg_sh.at[_didx(j)],
                                              dsem.at[nb]).wait()
                pltpu.async_copy(feat.at[_sidx(j + NBUF)], msgs.at[nb],
                                 gsem.at[nb])
        return 0
    lax.fori_loop(0, q_main // NSLOT, _group, 0)

    for b in range(NSLOT):
        pltpu.make_async_copy(msgs.at[b], agg_sh.at[_didx(0)],
                              ssem.at[b]).wait()
        if with_deg:
            pltpu.make_async_copy(ones_v, deg_sh.at[_didx(0)],
                                  dsem.at[b]).wait()

    def _tail_chunk(j):
        pltpu.async_copy(feat.at[_sidx(j)], msgs.at[0], gsem.at[0])
        pltpu.make_async_copy(feat.at[_sidx(j)], msgs.at[0],
                              gsem.at[0]).wait()
        pltpu.sync_copy(msgs.at[0], agg_sh.at[_didx(j)], add=True)
        if with_deg:
            pltpu.sync_copy(ones_v, deg_sh.at[_didx(j)], add=True)

    for t in range(q_main, q):
        _tail_chunk(t)
    if r:
        @pl.when(wid < r)
        def _():
            _tail_chunk(q)
    plsc.subcore_barrier()

    sl = pl.ds(sid * ROWS_PER_TILE, ROWS_PER_TILE)
    pltpu.sync_copy(agg_sh.at[sl], agg_out.at[cid, sl])
    if with_deg:
        pltpu.sync_copy(deg_sh.at[sl], deg_out.at[cid, sl])


def _make_sc_call(with_deg, n_rows):
    mesh = plsc.VectorSubcoreMesh(core_axis_name="c", subcore_axis_name="s")
    q = n_rows // 32
    if with_deg:
        out_type = [
            jax.ShapeDtypeStruct((2, N_PAD, D_HID), jnp.float32),
            jax.ShapeDtypeStruct((2, N_PAD), jnp.float32),
        ]
        scratch = [
            pltpu.VMEM(((q + 1) * CHUNK,), jnp.int32),
            pltpu.VMEM(((q + 1) * CHUNK,), jnp.int32),
            pltpu.VMEM((NSLOT, CHUNK, D_HID), jnp.float32),
            pltpu.VMEM((CHUNK,), jnp.float32),
            pltpu.VMEM((ROWS_PER_TILE, D_HID), jnp.float32),
            pltpu.VMEM((ROWS_PER_TILE,), jnp.float32),
            pltpu.VMEM_SHARED((N_PAD, D_HID), jnp.float32),
            pltpu.VMEM_SHARED((N_PAD,), jnp.float32),
            pltpu.SemaphoreType.DMA((NSLOT,)),
            pltpu.SemaphoreType.DMA((NSLOT,)),
            pltpu.SemaphoreType.DMA((NSLOT,)),
        ]
    else:
        out_type = [jax.ShapeDtypeStruct((2, N_PAD, D_HID), jnp.float32)]
        scratch = [
            pltpu.VMEM(((q + 1) * CHUNK,), jnp.int32),
            pltpu.VMEM(((q + 1) * CHUNK,), jnp.int32),
            pltpu.VMEM((NSLOT, CHUNK, D_HID), jnp.float32),
            pltpu.VMEM((ROWS_PER_TILE, D_HID), jnp.float32),
            pltpu.VMEM_SHARED((N_PAD, D_HID), jnp.float32),
            pltpu.SemaphoreType.DMA((NSLOT,)),
            pltpu.SemaphoreType.DMA((NSLOT,)),
        ]
    return pl.kernel(
        functools.partial(_sc_segsum_body, with_deg, n_rows),
        mesh=mesh,
        out_type=out_type,
        scratch_types=scratch,
        compiler_params=pltpu.CompilerParams(use_tc_tiling_on_sc=False),
    )


def _mm_body(x_ref, wl_ref, wr_ref, xl_ref, xr_ref):
    xb = x_ref[...]
    xl_ref[...] = jnp.dot(xb, wl_ref[...], preferred_element_type=jnp.float32)
    xr_ref[...] = jnp.dot(xb, wr_ref[...], preferred_element_type=jnp.float32)


def _mid_body(aggp_ref, degp_ref, xr_ref, b1l_ref, h_ref, dinv_ref):
    n = xr_ref.shape[0]
    deg = jnp.maximum(degp_ref[0, :n, :] + degp_ref[1, :n, :], 1.0)
    dinv = 1.0 / deg
    t = ((aggp_ref[0, :n, :] + aggp_ref[1, :n, :]) * dinv
         + b1l_ref[...] + xr_ref[...])
    nrm = jnp.sqrt(jnp.sum(t * t, axis=1, keepdims=True))
    t = t / jnp.maximum(nrm, 1e-12)
    h_ref[...] = jnp.maximum(t, 0.0)
    dinv_ref[...] = dinv


def _out_body(aggp_ref, dinv_ref, h_ref, w2l_ref, w2r_ref, b2l_ref, o_ref):
    n = h_ref.shape[0]
    agg2 = (aggp_ref[0, :n, :] + aggp_ref[1, :n, :]) * dinv_ref[...]
    z = (jnp.dot(agg2, w2l_ref[...], preferred_element_type=jnp.float32)
         + jnp.dot(h_ref[...], w2r_ref[...], preferred_element_type=jnp.float32)
         + b2l_ref[...])
    z = z - jnp.max(z, axis=1, keepdims=True)
    e = jnp.exp(z)
    o_ref[...] = e / jnp.sum(e, axis=1, keepdims=True)


def kernel(x, edge_index, W1l, b1l, W1r, W2l, b2l, W2r):
    n = x.shape[0]
    e_total = edge_index.shape[1]
    if e_total % CHUNK:
        pad_n = CHUNK - e_total % CHUNK
        ar = jnp.arange(pad_n, dtype=jnp.int32)
        src1d = jnp.concatenate([edge_index[0], ar % n])
        dst1d = jnp.concatenate([edge_index[1],
                                 N_NODES + ar % (N_PAD - N_NODES)])
    else:
        src1d = edge_index[0]
        dst1d = edge_index[1]
    n_rows = src1d.shape[0] // CHUNK

    xl, xr = pl.pallas_call(
        _mm_body,
        out_shape=[
            jax.ShapeDtypeStruct((n, D_HID), jnp.float32),
            jax.ShapeDtypeStruct((n, D_HID), jnp.float32),
        ],
    )(x, W1l.T, W1r.T)

    aggp1, degp = _make_sc_call(True, n_rows)(xl, src1d, dst1d)

    h, dinv = pl.pallas_call(
        _mid_body,
        out_shape=[
            jax.ShapeDtypeStruct((n, D_HID), jnp.float32),
            jax.ShapeDtypeStruct((n, 1), jnp.float32),
        ],
    )(aggp1, degp.reshape(2, N_PAD, 1), xr, b1l.reshape(1, D_HID))

    aggp2 = _make_sc_call(False, n_rows)(h, src1d, dst1d)
    if isinstance(aggp2, (list, tuple)):
        aggp2 = aggp2[0]

    out = pl.pallas_call(
        _out_body,
        out_shape=jax.ShapeDtypeStruct((n, D_OUT), jnp.float32),
    )(aggp2, dinv, h, W2l.T, W2r.T, b2l.reshape(1, D_OUT))
    return out


# speedup vs baseline: 23.4032x; 1.1990x over previous
import functools

import jax
import jax.numpy as jnp
from jax import lax
from jax.experimental import pallas as pl
from jax.experimental.pallas import tpu as pltpu
from jax.experimental.pallas import tpu_sc as plsc

N_NODES = 10000
N_PAD = 10240
D_IN = 128
D_HID = 16
D_OUT = 40
ROWS_PER_TILE = N_PAD // 16
CHUNK = 128
NBUF = 4
NSLOT = 8


def _sc_segsum_body(with_deg, n_rows, feat, edges, *rest):
    if with_deg:
        (agg_out, deg_out, edges_v, msgs, ones_v, zbuf, dzbuf,
         agg_sh, deg_sh, feat_sh, gsem, ssem, dsem) = rest
    else:
        agg_out, edges_v, msgs, zbuf, agg_sh, feat_sh, gsem, ssem = rest
    cid = lax.axis_index("c")
    sid = lax.axis_index("s")
    wid = cid * 16 + sid
    q, r = divmod(n_rows, 32)
    q_main = (q // NSLOT) * NSLOT
    base = wid * q

    def _zrow(i, _):
        zbuf[i, :] = jnp.zeros((16,), jnp.float32)
        return 0
    lax.fori_loop(0, ROWS_PER_TILE, _zrow, 0)
    pltpu.sync_copy(zbuf, agg_sh.at[pl.ds(sid * ROWS_PER_TILE, ROWS_PER_TILE)])
    if with_deg:
        def _zrow1(i, _):
            dzbuf[pl.ds(i * 16, 16)] = jnp.zeros((16,), jnp.float32)
            return 0
        lax.fori_loop(0, ROWS_PER_TILE // 16, _zrow1, 0)
        pltpu.sync_copy(dzbuf,
                        deg_sh.at[pl.ds(sid * ROWS_PER_TILE, ROWS_PER_TILE)])
        for j in range(CHUNK // 16):
            ones_v[pl.ds(j * 16, 16)] = jnp.ones((16,), jnp.float32)

    n_feat = feat.shape[0]
    f_rows = n_feat // 16
    fsl = pl.ds(sid * f_rows, f_rows)
    pltpu.sync_copy(feat.at[fsl], feat_sh.at[fsl])
    if n_feat % 16:
        @pl.when(sid == 15)
        def _():
            tail = pl.ds(16 * f_rows, n_feat - 16 * f_rows)
            pltpu.sync_copy(feat.at[tail], feat_sh.at[tail])
    plsc.subcore_barrier()

    pltpu.sync_copy(edges.at[pl.ds(base, q)], edges_v.at[pl.ds(0, q)])
    if r:
        @pl.when(wid < r)
        def _():
            pltpu.sync_copy(edges.at[pl.ds(32 * q + wid, 1)],
                            edges_v.at[pl.ds(q, 1)])

    def _sidx(j):
        return edges_v.at[j, 0]

    def _didx(j):
        return edges_v.at[j, 1]

    for b in range(NBUF):
        pltpu.async_copy(feat_sh.at[_sidx(b)], msgs.at[b], gsem.at[b])

    def _group(g, _):
        j0 = g * NSLOT
        for b in range(NSLOT):
            j = j0 + b
            pltpu.make_async_copy(feat_sh.at[_sidx(j)], msgs.at[b],
                                  gsem.at[b]).wait()
            pltpu.async_copy(msgs.at[b], agg_sh.at[_didx(j)], ssem.at[b],
                             add=True)
            if with_deg:
                pltpu.async_copy(ones_v, deg_sh.at[_didx(j)], dsem.at[b],
                                 add=True)
            nb = (b + NBUF) % NSLOT

            @pl.when(j + NBUF < q_main)
            def _():
                @pl.when(j + NBUF >= NSLOT)
                def _():
                    pltpu.make_async_copy(msgs.at[nb], agg_sh.at[_didx(j)],
                                          ssem.at[nb]).wait()
                    if with_deg:
                        pltpu.make_async_copy(ones_v, deg_sh.at[_didx(j)],
                                              dsem.at[nb]).wait()
                pltpu.async_copy(feat_sh.at[_sidx(j + NBUF)], msgs.at[nb],
                                 gsem.at[nb])
        return 0
    lax.fori_loop(0, q_main // NSLOT, _group, 0)

    for b in range(NSLOT):
        pltpu.make_async_copy(msgs.at[b], agg_sh.at[_didx(0)],
                              ssem.at[b]).wait()
        if with_deg:
            pltpu.make_async_copy(ones_v, deg_sh.at[_didx(0)],
                                  dsem.at[b]).wait()

    def _tail_chunk(j):
        pltpu.async_copy(feat_sh.at[_sidx(j)], msgs.at[0], gsem.at[0])
        pltpu.make_async_copy(feat_sh.at[_sidx(j)], msgs.at[0],
                              gsem.at[0]).wait()
        pltpu.sync_copy(msgs.at[0], agg_sh.at[_didx(j)], add=True)
        if with_deg:
            pltpu.sync_copy(ones_v, deg_sh.at[_didx(j)], add=True)

    for t in range(q_main, q):
        _tail_chunk(t)
    if r:
        @pl.when(wid < r)
        def _():
            _tail_chunk(q)
    plsc.subcore_barrier()

    sl = pl.ds(sid * ROWS_PER_TILE, ROWS_PER_TILE)
    pltpu.sync_copy(agg_sh.at[sl], agg_out.at[cid, sl])
    if with_deg:
        pltpu.sync_copy(deg_sh.at[sl], deg_out.at[cid, sl])


def _make_sc_call(with_deg, n_rows, n_feat):
    mesh = plsc.VectorSubcoreMesh(core_axis_name="c", subcore_axis_name="s")
    q = n_rows // 32
    if with_deg:
        out_type = [
            jax.ShapeDtypeStruct((2, N_PAD, D_HID), jnp.float32),
            jax.ShapeDtypeStruct((2, N_PAD), jnp.float32),
        ]
        scratch = [
            pltpu.VMEM((q + 1, 2, CHUNK), jnp.int32),
            pltpu.VMEM((NSLOT, CHUNK, D_HID), jnp.float32),
            pltpu.VMEM((CHUNK,), jnp.float32),
            pltpu.VMEM((ROWS_PER_TILE, D_HID), jnp.float32),
            pltpu.VMEM((ROWS_PER_TILE,), jnp.float32),
            pltpu.VMEM_SHARED((N_PAD, D_HID), jnp.float32),
            pltpu.VMEM_SHARED((N_PAD,), jnp.float32),
            pltpu.VMEM_SHARED((n_feat, D_HID), jnp.float32),
            pltpu.SemaphoreType.DMA((NSLOT,)),
            pltpu.SemaphoreType.DMA((NSLOT,)),
            pltpu.SemaphoreType.DMA((NSLOT,)),
        ]
    else:
        out_type = [jax.ShapeDtypeStruct((2, N_PAD, D_HID), jnp.float32)]
        scratch = [
            pltpu.VMEM((q + 1, 2, CHUNK), jnp.int32),
            pltpu.VMEM((NSLOT, CHUNK, D_HID), jnp.float32),
            pltpu.VMEM((ROWS_PER_TILE, D_HID), jnp.float32),
            pltpu.VMEM_SHARED((N_PAD, D_HID), jnp.float32),
            pltpu.VMEM_SHARED((n_feat, D_HID), jnp.float32),
            pltpu.SemaphoreType.DMA((NSLOT,)),
            pltpu.SemaphoreType.DMA((NSLOT,)),
        ]
    return pl.kernel(
        functools.partial(_sc_segsum_body, with_deg, n_rows),
        mesh=mesh,
        out_type=out_type,
        scratch_types=scratch,
        compiler_params=pltpu.CompilerParams(use_tc_tiling_on_sc=False),
    )


def _mm_body(x_ref, wl_ref, wr_ref, xl_ref, xr_ref):
    xb = x_ref[...]
    xl_ref[...] = jnp.dot(xb, wl_ref[...], preferred_element_type=jnp.float32)
    xr_ref[...] = jnp.dot(xb, wr_ref[...], preferred_element_type=jnp.float32)


def _mid_body(aggp_ref, degp_ref, xr_ref, b1l_ref, h_ref, dinv_ref):
    n = xr_ref.shape[0]
    deg = jnp.maximum(degp_ref[0, :n, :] + degp_ref[1, :n, :], 1.0)
    dinv = 1.0 / deg
    t = ((aggp_ref[0, :n, :] + aggp_ref[1, :n, :]) * dinv
         + b1l_ref[...] + xr_ref[...])
    nrm = jnp.sqrt(jnp.sum(t * t, axis=1, keepdims=True))
    t = t / jnp.maximum(nrm, 1e-12)
    h_ref[...] = jnp.maximum(t, 0.0)
    dinv_ref[...] = dinv


def _out_body(aggp_ref, dinv_ref, h_ref, w2l_ref, w2r_ref, b2l_ref, o_ref):
    n = h_ref.shape[0]
    agg2 = (aggp_ref[0, :n, :] + aggp_ref[1, :n, :]) * dinv_ref[...]
    z = (jnp.dot(agg2, w2l_ref[...], preferred_element_type=jnp.float32)
         + jnp.dot(h_ref[...], w2r_ref[...], preferred_element_type=jnp.float32)
         + b2l_ref[...])
    z = z - jnp.max(z, axis=1, keepdims=True)
    e = jnp.exp(z)
    o_ref[...] = e / jnp.sum(e, axis=1, keepdims=True)


def kernel(x, edge_index, W1l, b1l, W1r, W2l, b2l, W2r):
    n = x.shape[0]
    e_total = edge_index.shape[1]
    if e_total % CHUNK:
        pad_n = CHUNK - e_total % CHUNK
        ar = jnp.arange(pad_n, dtype=jnp.int32)
        edge_index = jnp.concatenate(
            [edge_index,
             jnp.stack([ar % n, N_NODES + ar % (N_PAD - N_NODES)])], axis=1)
    edges = jnp.transpose(edge_index.reshape(2, -1, CHUNK), (1, 0, 2))
    n_rows = edges.shape[0]

    xl, xr = pl.pallas_call(
        _mm_body,
        out_shape=[
            jax.ShapeDtypeStruct((n, D_HID), jnp.float32),
            jax.ShapeDtypeStruct((n, D_HID), jnp.float32),
        ],
    )(x, W1l.T, W1r.T)

    aggp1, degp = _make_sc_call(True, n_rows, n)(xl, edges)

    h, dinv = pl.pallas_call(
        _mid_body,
        out_shape=[
            jax.ShapeDtypeStruct((n, D_HID), jnp.float32),
            jax.ShapeDtypeStruct((n, 1), jnp.float32),
        ],
    )(aggp1, degp.reshape(2, N_PAD, 1), xr, b1l.reshape(1, D_HID))

    aggp2 = _make_sc_call(False, n_rows, n)(h, edges)
    if isinstance(aggp2, (list, tuple)):
        aggp2 = aggp2[0]

    out = pl.pallas_call(
        _out_body,
        out_shape=jax.ShapeDtypeStruct((n, D_OUT), jnp.float32),
    )(aggp2, dinv, h, W2l.T, W2r.T, b2l.reshape(1, D_OUT))
    return out


# speedup vs baseline: 35.2170x; 1.5048x over previous
import functools

import jax
import jax.numpy as jnp
from jax import lax
from jax.experimental import pallas as pl
from jax.experimental.pallas import tpu as pltpu
from jax.experimental.pallas import tpu_sc as plsc

N_NODES = 10000
N_PAD = 10240
D_IN = 128
D_HID = 16
D_OUT = 40
ROWS_PER_TILE = N_PAD // 16
CHUNK = 128
NBUF = 4
NSLOT = 8


def _sc_segsum_body(with_deg, n_rows, feat, edges, *rest):
    if with_deg:
        (agg_out, deg_out, edges_v, msgs, ones_v, zbuf, dzbuf,
         agg_sh, deg_sh, feat_sh, gsem, ssem, dsem) = rest
    else:
        agg_out, edges_v, msgs, zbuf, agg_sh, feat_sh, gsem, ssem = rest
    cid = lax.axis_index("c")
    sid = lax.axis_index("s")
    wid = cid * 16 + sid
    q, r = divmod(n_rows, 32)
    q_main = (q // NSLOT) * NSLOT
    base = wid * q

    def _zrow(i, _):
        zbuf[i, :] = jnp.zeros((16,), jnp.float32)
        return 0
    lax.fori_loop(0, ROWS_PER_TILE, _zrow, 0)
    pltpu.sync_copy(zbuf, agg_sh.at[pl.ds(sid * ROWS_PER_TILE, ROWS_PER_TILE)])
    if with_deg:
        def _zrow1(i, _):
            dzbuf[pl.ds(i * 16, 16)] = jnp.zeros((16,), jnp.float32)
            return 0
        lax.fori_loop(0, ROWS_PER_TILE // 16, _zrow1, 0)
        pltpu.sync_copy(dzbuf,
                        deg_sh.at[pl.ds(sid * ROWS_PER_TILE, ROWS_PER_TILE)])
        for j in range(CHUNK // 16):
            ones_v[pl.ds(j * 16, 16)] = jnp.ones((16,), jnp.float32)

    n_feat = feat.shape[0]
    f_rows = n_feat // 16
    fsl = pl.ds(sid * f_rows, f_rows)
    pltpu.sync_copy(feat.at[fsl], feat_sh.at[fsl])
    if n_feat % 16:
        @pl.when(sid == 15)
        def _():
            tail = pl.ds(16 * f_rows, n_feat - 16 * f_rows)
            pltpu.sync_copy(feat.at[tail], feat_sh.at[tail])
    plsc.subcore_barrier()

    pltpu.sync_copy(edges.at[pl.ds(base, q)], edges_v.at[pl.ds(0, q)])
    if r:
        @pl.when(wid < r)
        def _():
            pltpu.sync_copy(edges.at[pl.ds(32 * q + wid, 1)],
                            edges_v.at[pl.ds(q, 1)])

    def _sidx(j):
        return edges_v.at[j, 0]

    def _didx(j):
        return edges_v.at[j, 1]

    for b in range(NBUF):
        pltpu.async_copy(feat_sh.at[_sidx(b)], msgs.at[b], gsem.at[b])

    def _group(g, _):
        j0 = g * NSLOT
        for b in range(NSLOT):
            j = j0 + b
            pltpu.make_async_copy(feat_sh.at[_sidx(j)], msgs.at[b],
                                  gsem.at[b]).wait()
            pltpu.async_copy(msgs.at[b], agg_sh.at[_didx(j)], ssem.at[b],
                             add=True)
            if with_deg:
                pltpu.async_copy(ones_v, deg_sh.at[_didx(j)], dsem.at[b],
                                 add=True)
            nb = (b + NBUF) % NSLOT

            @pl.when(j + NBUF < q_main)
            def _():
                @pl.when(j + NBUF >= NSLOT)
                def _():
                    pltpu.make_async_copy(msgs.at[nb], agg_sh.at[_didx(j)],
                                          ssem.at[nb]).wait()
                    if with_deg:
                        pltpu.make_async_copy(ones_v, deg_sh.at[_didx(j)],
                                              dsem.at[nb]).wait()
                pltpu.async_copy(feat_sh.at[_sidx(j + NBUF)], msgs.at[nb],
                                 gsem.at[nb])
        return 0
    lax.fori_loop(0, q_main // NSLOT, _group, 0)

    for b in range(NSLOT):
        pltpu.make_async_copy(msgs.at[b], agg_sh.at[_didx(0)],
                              ssem.at[b]).wait()
        if with_deg:
            pltpu.make_async_copy(ones_v, deg_sh.at[_didx(0)],
                                  dsem.at[b]).wait()

    def _tail_chunk(j):
        pltpu.async_copy(feat_sh.at[_sidx(j)], msgs.at[0], gsem.at[0])
        pltpu.make_async_copy(feat_sh.at[_sidx(j)], msgs.at[0],
                              gsem.at[0]).wait()
        pltpu.sync_copy(msgs.at[0], agg_sh.at[_didx(j)], add=True)
        if with_deg:
            pltpu.sync_copy(ones_v, deg_sh.at[_didx(j)], add=True)

    for t in range(q_main, q):
        _tail_chunk(t)
    if r:
        @pl.when(wid < r)
        def _():
            _tail_chunk(q)
    plsc.subcore_barrier()

    sl = pl.ds(sid * ROWS_PER_TILE, ROWS_PER_TILE)
    pltpu.sync_copy(agg_sh.at[sl], agg_out.at[cid, sl])
    if with_deg:
        pltpu.sync_copy(deg_sh.at[sl], deg_out.at[cid, sl])


def _make_sc_call(with_deg, n_rows, n_feat):
    mesh = plsc.VectorSubcoreMesh(core_axis_name="c", subcore_axis_name="s")
    q = n_rows // 32
    if with_deg:
        out_type = [
            jax.ShapeDtypeStruct((2, N_PAD, D_HID), jnp.float32),
            jax.ShapeDtypeStruct((2, N_PAD), jnp.float32),
        ]
        scratch = [
            pltpu.VMEM((q + 1, 2, CHUNK), jnp.int32),
            pltpu.VMEM((NSLOT, CHUNK, D_HID), jnp.float32),
            pltpu.VMEM((CHUNK,), jnp.float32),
            pltpu.VMEM((ROWS_PER_TILE, D_HID), jnp.float32),
            pltpu.VMEM((ROWS_PER_TILE,), jnp.float32),
            pltpu.VMEM_SHARED((N_PAD, D_HID), jnp.float32),
            pltpu.VMEM_SHARED((N_PAD,), jnp.float32),
            pltpu.VMEM_SHARED((n_feat, D_HID), jnp.float32),
            pltpu.SemaphoreType.DMA((NSLOT,)),
            pltpu.SemaphoreType.DMA((NSLOT,)),
            pltpu.SemaphoreType.DMA((NSLOT,)),
        ]
    else:
        out_type = [jax.ShapeDtypeStruct((2, N_PAD, D_HID), jnp.float32)]
        scratch = [
            pltpu.VMEM((q + 1, 2, CHUNK), jnp.int32),
            pltpu.VMEM((NSLOT, CHUNK, D_HID), jnp.float32),
            pltpu.VMEM((ROWS_PER_TILE, D_HID), jnp.float32),
            pltpu.VMEM_SHARED((N_PAD, D_HID), jnp.float32),
            pltpu.VMEM_SHARED((n_feat, D_HID), jnp.float32),
            pltpu.SemaphoreType.DMA((NSLOT,)),
            pltpu.SemaphoreType.DMA((NSLOT,)),
        ]
    return pl.kernel(
        functools.partial(_sc_segsum_body, with_deg, n_rows),
        mesh=mesh,
        out_type=out_type,
        scratch_types=scratch,
        compiler_params=pltpu.CompilerParams(use_tc_tiling_on_sc=False),
    )


def _mm_body(x3_ref, tl_ref, tr_ref, xl_ref, xr_ref):
    acc_l = acc_r = 0.0
    for a in range(8):
        xb = x3_ref[:, a, :]
        acc_l += jnp.dot(xb, tl_ref[a], preferred_element_type=jnp.float32)
        acc_r += jnp.dot(xb, tr_ref[a], preferred_element_type=jnp.float32)
    xl_ref[...] = acc_l
    xr_ref[...] = acc_r


def _mid_body(aggp_ref, degp_ref, xr_ref, b1lt_ref, st8_ref, s16_ref,
              h_ref, dinv_ref):
    m = xr_ref.shape[0]
    deg8 = jnp.maximum(degp_ref[0, :m, :] + degp_ref[1, :m, :], 1.0)
    dinv8 = 1.0 / deg8
    dinvp = jnp.dot(dinv8, st8_ref[...], preferred_element_type=jnp.float32)
    t = ((aggp_ref[0, :m, :] + aggp_ref[1, :m, :]) * dinvp
         + b1lt_ref[...] + xr_ref[...])
    nrm2 = jnp.dot(t * t, s16_ref[...], preferred_element_type=jnp.float32)
    rn8 = 1.0 / jnp.maximum(jnp.sqrt(nrm2), 1e-12)
    rnp = jnp.dot(rn8, st8_ref[...], preferred_element_type=jnp.float32)
    h_ref[...] = jnp.maximum(t * rnp, 0.0)
    dinv_ref[...] = dinv8


def _out_body(aggp_ref, dinv_ref, h_ref, st8_ref, w2l_ref, w2r_ref,
              b2lt_ref, s40_ref, st40_ref, o_ref):
    m = h_ref.shape[0]
    dinvp = jnp.dot(dinv_ref[...], st8_ref[...],
                    preferred_element_type=jnp.float32)
    agg2 = (aggp_ref[0, :m, :] + aggp_ref[1, :m, :]) * dinvp
    z = (jnp.dot(agg2, w2l_ref[...], preferred_element_type=jnp.float32)
         + jnp.dot(h_ref[...], w2r_ref[...], preferred_element_type=jnp.float32)
         + b2lt_ref[...])
    e = jnp.exp(z)
    s8 = jnp.dot(e, s40_ref[...], preferred_element_type=jnp.float32)
    o_ref[...] = e * jnp.dot(1.0 / s8, st40_ref[...],
                             preferred_element_type=jnp.float32)


def kernel(x, edge_index, W1l, b1l, W1r, W2l, b2l, W2r):
    n = x.shape[0]
    m = n // 8
    e_total = edge_index.shape[1]
    if e_total % CHUNK:
        pad_n = CHUNK - e_total % CHUNK
        ar = jnp.arange(pad_n, dtype=jnp.int32)
        edge_index = jnp.concatenate(
            [edge_index,
             jnp.stack([ar % n, N_NODES + ar % (N_PAD - N_NODES)])], axis=1)
    edges = jnp.transpose(edge_index.reshape(2, -1, CHUNK), (1, 0, 2))
    n_rows = edges.shape[0]

    eye8 = jnp.eye(8, dtype=jnp.float32)
    tl = (eye8[:, None, :, None] * W1l.T[None, :, None, :]).reshape(8, 128, 128)
    tr = (eye8[:, None, :, None] * W1r.T[None, :, None, :]).reshape(8, 128, 128)
    st8 = jnp.kron(eye8, jnp.ones((1, D_HID), jnp.float32))
    s16 = jnp.kron(eye8, jnp.ones((D_HID, 1), jnp.float32))
    w2lb = jnp.kron(eye8, W2l.T)
    w2rb = jnp.kron(eye8, W2r.T)
    s40 = jnp.kron(eye8, jnp.ones((D_OUT, 1), jnp.float32))
    st40 = jnp.kron(eye8, jnp.ones((1, D_OUT), jnp.float32))
    b1lt = jnp.tile(b1l, 8).reshape(1, 8 * D_HID)
    b2lt = jnp.tile(b2l, 8).reshape(1, 8 * D_OUT)

    x3 = x.reshape(m, 8, D_IN)
    xl_p, xr_p = pl.pallas_call(
        _mm_body,
        out_shape=[
            jax.ShapeDtypeStruct((m, 128), jnp.float32),
            jax.ShapeDtypeStruct((m, 128), jnp.float32),
        ],
    )(x3, tl, tr)

    aggp1, degp = _make_sc_call(True, n_rows, n)(
        xl_p.reshape(n, D_HID), edges)

    h_p, dinv8 = pl.pallas_call(
        _mid_body,
        out_shape=[
            jax.ShapeDtypeStruct((m, 128), jnp.float32),
            jax.ShapeDtypeStruct((m, 8), jnp.float32),
        ],
    )(aggp1.reshape(2, N_PAD // 8, 128), degp.reshape(2, N_PAD // 8, 8),
      xr_p, b1lt, st8, s16)

    aggp2 = _make_sc_call(False, n_rows, n)(h_p.reshape(n, D_HID), edges)
    if isinstance(aggp2, (list, tuple)):
        aggp2 = aggp2[0]

    out_p = pl.pallas_call(
        _out_body,
        out_shape=jax.ShapeDtypeStruct((m, 8 * D_OUT), jnp.float32),
    )(aggp2.reshape(2, N_PAD // 8, 128), dinv8, h_p, st8, w2lb, w2rb,
      b2lt, s40, st40)
    return out_p.reshape(n, D_OUT)


# speedup vs baseline: 35.2402x; 1.0007x over previous
import functools

import jax
import jax.numpy as jnp
from jax import lax
from jax.experimental import pallas as pl
from jax.experimental.pallas import tpu as pltpu
from jax.experimental.pallas import tpu_sc as plsc

N_NODES = 10000
N_PAD = 10240
D_IN = 128
D_HID = 16
D_OUT = 40
ROWS_PER_TILE = N_PAD // 16
CHUNK = 128
NBUF = 4
NSLOT = 8


def _sc_segsum_body(with_deg, n_rows, feat, edges, *rest):
    if with_deg:
        (agg_out, deg_out, edges_v, msgs, ones_v, zbuf, dzbuf,
         agg_sh, deg_sh, feat_sh, gsem, ssem, dsem) = rest
    else:
        agg_out, edges_v, msgs, zbuf, agg_sh, feat_sh, gsem, ssem = rest
    cid = lax.axis_index("c")
    sid = lax.axis_index("s")
    wid = cid * 16 + sid
    q, r = divmod(n_rows, 32)
    q_main = (q // NSLOT) * NSLOT
    base = wid * q

    def _zrow(i, _):
        zbuf[i, :] = jnp.zeros((16,), jnp.float32)
        return 0
    lax.fori_loop(0, ROWS_PER_TILE, _zrow, 0)
    pltpu.sync_copy(zbuf, agg_sh.at[pl.ds(sid * ROWS_PER_TILE, ROWS_PER_TILE)])
    if with_deg:
        def _zrow1(i, _):
            dzbuf[pl.ds(i * 16, 16)] = jnp.zeros((16,), jnp.float32)
            return 0
        lax.fori_loop(0, ROWS_PER_TILE // 16, _zrow1, 0)
        pltpu.sync_copy(dzbuf,
                        deg_sh.at[pl.ds(sid * ROWS_PER_TILE, ROWS_PER_TILE)])
        for j in range(CHUNK // 16):
            ones_v[pl.ds(j * 16, 16)] = jnp.ones((16,), jnp.float32)

    n_feat = feat.shape[0]
    f_rows = n_feat // 16
    fsl = pl.ds(sid * f_rows, f_rows)
    pltpu.sync_copy(feat.at[fsl], feat_sh.at[fsl])
    if n_feat % 16:
        @pl.when(sid == 15)
        def _():
            tail = pl.ds(16 * f_rows, n_feat - 16 * f_rows)
            pltpu.sync_copy(feat.at[tail], feat_sh.at[tail])
    plsc.subcore_barrier()

    pltpu.sync_copy(edges.at[pl.ds(base, q)], edges_v.at[pl.ds(0, q)])
    if r:
        @pl.when(wid < r)
        def _():
            pltpu.sync_copy(edges.at[pl.ds(32 * q + wid, 1)],
                            edges_v.at[pl.ds(q, 1)])

    def _sidx(j):
        return edges_v.at[j, 0]

    def _didx(j):
        return edges_v.at[j, 1]

    for b in range(NBUF):
        pltpu.async_copy(feat_sh.at[_sidx(b)], msgs.at[b], gsem.at[b])

    def _group(g, _):
        j0 = g * NSLOT
        for b in range(NSLOT):
            j = j0 + b
            pltpu.make_async_copy(feat_sh.at[_sidx(j)], msgs.at[b],
                                  gsem.at[b]).wait()
            pltpu.async_copy(msgs.at[b], agg_sh.at[_didx(j)], ssem.at[b],
                             add=True)
            if with_deg:
                pltpu.async_copy(ones_v, deg_sh.at[_didx(j)], dsem.at[b],
                                 add=True)
            nb = (b + NBUF) % NSLOT

            @pl.when(j + NBUF < q_main)
            def _():
                @pl.when(j + NBUF >= NSLOT)
                def _():
                    pltpu.make_async_copy(msgs.at[nb], agg_sh.at[_didx(j)],
                                          ssem.at[nb]).wait()
                    if with_deg:
                        pltpu.make_async_copy(ones_v, deg_sh.at[_didx(j)],
                                              dsem.at[nb]).wait()
                pltpu.async_copy(feat_sh.at[_sidx(j + NBUF)], msgs.at[nb],
                                 gsem.at[nb])
        return 0
    lax.fori_loop(0, q_main // NSLOT, _group, 0)

    for b in range(NSLOT):
        pltpu.make_async_copy(msgs.at[b], agg_sh.at[_didx(0)],
                              ssem.at[b]).wait()
        if with_deg:
            pltpu.make_async_copy(ones_v, deg_sh.at[_didx(0)],
                                  dsem.at[b]).wait()

    def _tail_chunk(j):
        pltpu.async_copy(feat_sh.at[_sidx(j)], msgs.at[0], gsem.at[0])
        pltpu.make_async_copy(feat_sh.at[_sidx(j)], msgs.at[0],
                              gsem.at[0]).wait()
        pltpu.sync_copy(msgs.at[0], agg_sh.at[_didx(j)], add=True)
        if with_deg:
            pltpu.sync_copy(ones_v, deg_sh.at[_didx(j)], add=True)

    for t in range(q_main, q):
        _tail_chunk(t)
    if r:
        @pl.when(wid < r)
        def _():
            _tail_chunk(q)
    plsc.subcore_barrier()

    sl = pl.ds(sid * ROWS_PER_TILE, ROWS_PER_TILE)
    pltpu.sync_copy(agg_sh.at[sl], agg_out.at[cid, sl])
    if with_deg:
        pltpu.sync_copy(deg_sh.at[sl], deg_out.at[cid, sl])


def _make_sc_call(with_deg, n_rows, n_feat):
    mesh = plsc.VectorSubcoreMesh(core_axis_name="c", subcore_axis_name="s")
    q = n_rows // 32
    if with_deg:
        out_type = [
            jax.ShapeDtypeStruct((2, N_PAD, D_HID), jnp.float32),
            jax.ShapeDtypeStruct((2, N_PAD), jnp.float32),
        ]
        scratch = [
            pltpu.VMEM((q + 1, 2, CHUNK), jnp.int32),
            pltpu.VMEM((NSLOT, CHUNK, D_HID), jnp.float32),
            pltpu.VMEM((CHUNK,), jnp.float32),
            pltpu.VMEM((ROWS_PER_TILE, D_HID), jnp.float32),
            pltpu.VMEM((ROWS_PER_TILE,), jnp.float32),
            pltpu.VMEM_SHARED((N_PAD, D_HID), jnp.float32),
            pltpu.VMEM_SHARED((N_PAD,), jnp.float32),
            pltpu.VMEM_SHARED((n_feat, D_HID), jnp.float32),
            pltpu.SemaphoreType.DMA((NSLOT,)),
            pltpu.SemaphoreType.DMA((NSLOT,)),
            pltpu.SemaphoreType.DMA((NSLOT,)),
        ]
    else:
        out_type = [jax.ShapeDtypeStruct((2, N_PAD, D_HID), jnp.float32)]
        scratch = [
            pltpu.VMEM((q + 1, 2, CHUNK), jnp.int32),
            pltpu.VMEM((NSLOT, CHUNK, D_HID), jnp.float32),
            pltpu.VMEM((ROWS_PER_TILE, D_HID), jnp.float32),
            pltpu.VMEM_SHARED((N_PAD, D_HID), jnp.float32),
            pltpu.VMEM_SHARED((n_feat, D_HID), jnp.float32),
            pltpu.SemaphoreType.DMA((NSLOT,)),
            pltpu.SemaphoreType.DMA((NSLOT,)),
        ]
    return pl.kernel(
        functools.partial(_sc_segsum_body, with_deg, n_rows),
        mesh=mesh,
        out_type=out_type,
        scratch_types=scratch,
        compiler_params=pltpu.CompilerParams(use_tc_tiling_on_sc=False),
    )


def _mm_body(x3_ref, tl_ref, tr_ref, xl_ref, xr_ref):
    acc_l = acc_r = 0.0
    for a in range(8):
        xb = x3_ref[:, a, :]
        acc_l += jnp.dot(xb, tl_ref[a], preferred_element_type=jnp.float32)
        acc_r += jnp.dot(xb, tr_ref[a], preferred_element_type=jnp.float32)
    xl_ref[...] = acc_l
    xr_ref[...] = acc_r


def _mid_body(aggp_ref, degp_ref, xr_ref, b1lt_ref, st8_ref, s16_ref,
              h_ref, dinv_ref):
    m = xr_ref.shape[0]
    deg8 = jnp.maximum(degp_ref[0, :m, :] + degp_ref[1, :m, :], 1.0)
    dinv8 = 1.0 / deg8
    dinvp = jnp.dot(dinv8, st8_ref[...], preferred_element_type=jnp.float32)
    t = ((aggp_ref[0, :m, :] + aggp_ref[1, :m, :]) * dinvp
         + b1lt_ref[...] + xr_ref[...])
    nrm2 = jnp.dot(t * t, s16_ref[...], preferred_element_type=jnp.float32)
    rn8 = 1.0 / jnp.maximum(jnp.sqrt(nrm2), 1e-12)
    rnp = jnp.dot(rn8, st8_ref[...], preferred_element_type=jnp.float32)
    h_ref[...] = jnp.maximum(t * rnp, 0.0)
    dinv_ref[...] = dinv8


def _out_body(aggp_ref, dinv_ref, h_ref, st8_ref, w2l_ref, w2r_ref,
              b2lt_ref, s40_ref, st40_ref, o_ref):
    m = h_ref.shape[0]
    dinvp = jnp.dot(dinv_ref[...], st8_ref[...],
                    preferred_element_type=jnp.float32)
    agg2 = (aggp_ref[0, :m, :] + aggp_ref[1, :m, :]) * dinvp
    z = (jnp.dot(agg2, w2l_ref[...], preferred_element_type=jnp.float32)
         + jnp.dot(h_ref[...], w2r_ref[...], preferred_element_type=jnp.float32)
         + b2lt_ref[...])
    e = jnp.exp(z - jnp.max(z, axis=1, keepdims=True))
    s8 = jnp.dot(e, s40_ref[...], preferred_element_type=jnp.float32)
    o_ref[...] = e * jnp.dot(1.0 / s8, st40_ref[...],
                             preferred_element_type=jnp.float32)


def kernel(x, edge_index, W1l, b1l, W1r, W2l, b2l, W2r):
    n = x.shape[0]
    m = n // 8
    e_total = edge_index.shape[1]
    if e_total % CHUNK:
        pad_n = CHUNK - e_total % CHUNK
        ar = jnp.arange(pad_n, dtype=jnp.int32)
        edge_index = jnp.concatenate(
            [edge_index,
             jnp.stack([ar % n, N_NODES + ar % (N_PAD - N_NODES)])], axis=1)
    edges = jnp.transpose(edge_index.reshape(2, -1, CHUNK), (1, 0, 2))
    n_rows = edges.shape[0]

    eye8 = jnp.eye(8, dtype=jnp.float32)
    tl = (eye8[:, None, :, None] * W1l.T[None, :, None, :]).reshape(8, 128, 128)
    tr = (eye8[:, None, :, None] * W1r.T[None, :, None, :]).reshape(8, 128, 128)
    st8 = jnp.kron(eye8, jnp.ones((1, D_HID), jnp.float32))
    s16 = jnp.kron(eye8, jnp.ones((D_HID, 1), jnp.float32))
    w2lb = jnp.kron(eye8, W2l.T)
    w2rb = jnp.kron(eye8, W2r.T)
    s40 = jnp.kron(eye8, jnp.ones((D_OUT, 1), jnp.float32))
    st40 = jnp.kron(eye8, jnp.ones((1, D_OUT), jnp.float32))
    b1lt = jnp.tile(b1l, 8).reshape(1, 8 * D_HID)
    b2lt = jnp.tile(b2l, 8).reshape(1, 8 * D_OUT)

    x3 = x.reshape(m, 8, D_IN)
    xl_p, xr_p = pl.pallas_call(
        _mm_body,
        out_shape=[
            jax.ShapeDtypeStruct((m, 128), jnp.float32),
            jax.ShapeDtypeStruct((m, 128), jnp.float32),
        ],
    )(x3, tl, tr)

    aggp1, degp = _make_sc_call(True, n_rows, n)(
        xl_p.reshape(n, D_HID), edges)

    h_p, dinv8 = pl.pallas_call(
        _mid_body,
        out_shape=[
            jax.ShapeDtypeStruct((m, 128), jnp.float32),
            jax.ShapeDtypeStruct((m, 8), jnp.float32),
        ],
    )(aggp1.reshape(2, N_PAD // 8, 128), degp.reshape(2, N_PAD // 8, 8),
      xr_p, b1lt, st8, s16)

    aggp2 = _make_sc_call(False, n_rows, n)(h_p.reshape(n, D_HID), edges)
    if isinstance(aggp2, (list, tuple)):
        aggp2 = aggp2[0]

    out_p = pl.pallas_call(
        _out_body,
        out_shape=jax.ShapeDtypeStruct((m, 8 * D_OUT), jnp.float32),
    )(aggp2.reshape(2, N_PAD // 8, 128), dinv8, h_p, st8, w2lb, w2rb,
      b2lt, s40, st40)
    return out_p.reshape(n, D_OUT)
